# Initial kernel scaffold; baseline (speedup 1.0000x reference)
import jax
import jax.numpy as jnp
from jax.experimental import pallas as pl


def kernel(x, edge_attr, edge_index, eW1, eb1, eW2, eb2, eW3, eb3, eg, ebeta, nW1, nb1, nW2, nb2, nW3, nb3, ng, nbeta):
    raise NotImplementedError("write your pallas kernel here")


# speedup vs baseline: 1.0852x; 1.0852x over previous
import functools

import jax
import jax.numpy as jnp
from jax import lax
from jax.experimental import pallas as pl
from jax.experimental.pallas import tpu as pltpu

_N = 10000
_E = 160000
_D = 128
_HALF = _D // 2

_BN = 1000
_BE = 2000

_INTERPRET = False


def _silu(v):
    return v * jax.nn.sigmoid(v)


def _ln(v, g, b):
    m = jnp.mean(v, axis=-1, keepdims=True)
    c = v - m
    var = jnp.mean(c * c, axis=-1, keepdims=True)
    return c * lax.rsqrt(var + 1e-5) * g + b


def _pre_body(x_ref, wa_ref, wb_ref, oa_ref, ob_ref):
    xv = x_ref[...]
    oa_ref[...] = jnp.dot(xv, wa_ref[...], preferred_element_type=jnp.float32)
    ob_ref[...] = jnp.dot(xv, wb_ref[...], preferred_element_type=jnp.float32)


def _premul(x, w1a, w1b):
    grid = _N // _BN
    return pl.pallas_call(
        _pre_body,
        grid=(grid,),
        in_specs=[
            pl.BlockSpec((_BN, _D), lambda i: (i, 0)),
            pl.BlockSpec((_D, _D), lambda i: (0, 0)),
            pl.BlockSpec((_D, _D), lambda i: (0, 0)),
        ],
        out_specs=[
            pl.BlockSpec((_BN, _D), lambda i: (i, 0)),
            pl.BlockSpec((_BN, _D), lambda i: (i, 0)),
        ],
        out_shape=[
            jax.ShapeDtypeStruct((_N, _D), jnp.float32),
            jax.ShapeDtypeStruct((_N, _D), jnp.float32),
        ],
        interpret=_INTERPRET,
    )(x, w1a, w1b)


def _edge_body(g1_ref, g2_ref, ea_ref, w1c_ref, b1_ref, w2_ref, b2_ref,
               w3_ref, b3_ref, eg_ref, ebeta_ref, oe_ref, hd_ref, hs_ref):
    ea = ea_ref[...]
    pre1 = (g1_ref[...] + g2_ref[...] + b1_ref[...]
            + jnp.dot(ea, w1c_ref[...], preferred_element_type=jnp.float32))
    h = _silu(pre1)
    h = _silu(jnp.dot(h, w2_ref[...], preferred_element_type=jnp.float32)
              + b2_ref[...])
    e3 = jnp.dot(h, w3_ref[...], preferred_element_type=jnp.float32) + b3_ref[...]
    ne = _ln(e3, eg_ref[...], ebeta_ref[...])
    oe_ref[...] = ne + ea
    hd_ref[...] = ne[:, :_HALF]
    hs_ref[...] = ne[:, _HALF:]


def _edge_mlp(g1, g2, ea, w1c, b1, w2, b2, w3, b3, eg, ebeta):
    grid = _E // _BE
    row = lambda i: (i, 0)
    full = lambda i: (0, 0)
    return pl.pallas_call(
        _edge_body,
        grid=(grid,),
        in_specs=[
            pl.BlockSpec((_BE, _D), row),
            pl.BlockSpec((_BE, _D), row),
            pl.BlockSpec((_BE, _D), row),
            pl.BlockSpec((_D, _D), full),
            pl.BlockSpec((1, _D), full),
            pl.BlockSpec((_D, _D), full),
            pl.BlockSpec((1, _D), full),
            pl.BlockSpec((_D, _D), full),
            pl.BlockSpec((1, _D), full),
            pl.BlockSpec((1, _D), full),
            pl.BlockSpec((1, _D), full),
        ],
        out_specs=[
            pl.BlockSpec((_BE, _D), row),
            pl.BlockSpec((_BE, _HALF), row),
            pl.BlockSpec((_BE, _HALF), row),
        ],
        out_shape=[
            jax.ShapeDtypeStruct((_E, _D), jnp.float32),
            jax.ShapeDtypeStruct((_E, _HALF), jnp.float32),
            jax.ShapeDtypeStruct((_E, _HALF), jnp.float32),
        ],
        interpret=_INTERPRET,
    )(g1, g2, ea, w1c, b1, w2, b2, w3, b3, eg, ebeta)


def _node_body(x_ref, p0_ref, p1_ref, w1a_ref, w1b_ref, b1_ref, w2_ref,
               b2_ref, w3_ref, b3_ref, ng_ref, nbeta_ref, ox_ref):
    xv = x_ref[...]
    agg = p0_ref[...] + p1_ref[...]
    pre1 = (jnp.dot(xv, w1a_ref[...], preferred_element_type=jnp.float32)
            + jnp.dot(agg, w1b_ref[...], preferred_element_type=jnp.float32)
            + b1_ref[...])
    h = _silu(pre1)
    h = _silu(jnp.dot(h, w2_ref[...], preferred_element_type=jnp.float32)
              + b2_ref[...])
    e3 = jnp.dot(h, w3_ref[...], preferred_element_type=jnp.float32) + b3_ref[...]
    ox_ref[...] = _ln(e3, ng_ref[...], nbeta_ref[...]) + xv


def _node_mlp(x, p0, p1, w1a, w1b, b1, w2, b2, w3, b3, ng, nbeta):
    grid = _N // _BN
    row = lambda i: (i, 0)
    full = lambda i: (0, 0)
    return pl.pallas_call(
        _node_body,
        grid=(grid,),
        in_specs=[
            pl.BlockSpec((_BN, _D), row),
            pl.BlockSpec((_BN, _HALF), row),
            pl.BlockSpec((_BN, _HALF), row),
            pl.BlockSpec((_D, _D), full),
            pl.BlockSpec((_HALF, _D), full),
            pl.BlockSpec((1, _D), full),
            pl.BlockSpec((_D, _D), full),
            pl.BlockSpec((1, _D), full),
            pl.BlockSpec((_D, _D), full),
            pl.BlockSpec((1, _D), full),
            pl.BlockSpec((1, _D), full),
            pl.BlockSpec((1, _D), full),
        ],
        out_specs=pl.BlockSpec((_BN, _D), row),
        out_shape=jax.ShapeDtypeStruct((_N, _D), jnp.float32),
        interpret=_INTERPRET,
    )(x, p0, p1, w1a, w1b, b1, w2, b2, w3, b3, ng, nbeta)


def kernel(x, edge_attr, edge_index,
           eW1, eb1, eW2, eb2, eW3, eb3, eg, ebeta,
           nW1, nb1, nW2, nb2, nW3, nb3, ng, nbeta):
    src = edge_index[0].astype(jnp.int32)
    dst = edge_index[1].astype(jnp.int32)
    w1a, w1b, w1c = eW1[:_D], eW1[_D:2 * _D], eW1[2 * _D:]
    r = lambda v: v.reshape(1, _D)

    xs1, xd1 = _premul(x, w1a, w1b)

    g1 = jnp.take(xs1, src, axis=0)
    g2 = jnp.take(xd1, dst, axis=0)

    out_edge, hd, hs = _edge_mlp(g1, g2, edge_attr, w1c, r(eb1), eW2, r(eb2),
                                 eW3, r(eb3), r(eg), r(ebeta))

    p0 = jax.ops.segment_sum(hd, dst, num_segments=_N)
    p1 = jax.ops.segment_sum(hs, src, num_segments=_N)

    nw1a, nw1b = nW1[:_D], nW1[_D:]
    rh = lambda v: v.reshape(1, _D)
    out_x = _node_mlp(x, p0, p1, nw1a, nw1b, rh(nb1), nW2, rh(nb2),
                      nW3, rh(nb3), rh(ng), rh(nbeta))
    return (out_x, out_edge)


# speedup vs baseline: 2.7865x; 2.5678x over previous
import functools

import jax
import jax.numpy as jnp
from jax import lax
from jax.experimental import pallas as pl
from jax.experimental.pallas import tpu as pltpu
from jax.experimental.pallas import tpu_sc as plsc

_N = 10000
_E = 160000
_D = 128
_HALF = _D // 2

_BN = 1000
_BE = 2000

_NC = 2
_NS = 16
_NW = _NC * _NS
_PERW = _E // _NW
_CH = 100
_NCH = _PERW // _CH

_INTERPRET = False


def _silu(v):
    return v * jax.nn.sigmoid(v)


def _ln(v, g, b):
    m = jnp.mean(v, axis=-1, keepdims=True)
    c = v - m
    var = jnp.mean(c * c, axis=-1, keepdims=True)
    return c * lax.rsqrt(var + 1e-5) * g + b


def _pre_body(x_ref, wa_ref, wb_ref, oa_ref, ob_ref):
    xv = x_ref[...]
    oa_ref[...] = jnp.dot(xv, wa_ref[...], preferred_element_type=jnp.float32)
    ob_ref[...] = jnp.dot(xv, wb_ref[...], preferred_element_type=jnp.float32)


def _premul(x, w1a, w1b):
    grid = _N // _BN
    return pl.pallas_call(
        _pre_body,
        grid=(grid,),
        in_specs=[
            pl.BlockSpec((_BN, _D), lambda i: (i, 0)),
            pl.BlockSpec((_D, _D), lambda i: (0, 0)),
            pl.BlockSpec((_D, _D), lambda i: (0, 0)),
        ],
        out_specs=[
            pl.BlockSpec((_BN, _D), lambda i: (i, 0)),
            pl.BlockSpec((_BN, _D), lambda i: (i, 0)),
        ],
        out_shape=[
            jax.ShapeDtypeStruct((_N, _D), jnp.float32),
            jax.ShapeDtypeStruct((_N, _D), jnp.float32),
        ],
        interpret=_INTERPRET,
    )(x, w1a, w1b)


def _gather_body(xs1_hbm, xd1_hbm, src_hbm, dst_hbm, g1_hbm, g2_hbm,
                 idx_s, idx_d, ra0, ra1, rb0, rb1,
                 sa0, sa1, sb0, sb1):
    wid = lax.axis_index("s") * _NC + lax.axis_index("c")
    pltpu.sync_copy(src_hbm.at[wid], idx_s)
    pltpu.sync_copy(dst_hbm.at[wid], idx_d)
    ra = (ra0, ra1)
    rb = (rb0, rb1)
    sa = (sa0, sa1)
    sb = (sb0, sb1)

    def fire(j, p):
        pltpu.make_async_copy(xs1_hbm.at[idx_s.at[j]], ra[p], sa[p]).start()
        pltpu.make_async_copy(xd1_hbm.at[idx_d.at[j]], rb[p], sb[p]).start()

    def drain_and_write(j, p):
        pltpu.make_async_copy(xs1_hbm.at[idx_s.at[j]], ra[p], sa[p]).wait()
        pltpu.make_async_copy(xd1_hbm.at[idx_d.at[j]], rb[p], sb[p]).wait()
        cid = wid * _NCH + j
        pltpu.sync_copy(ra[p], g1_hbm.at[cid])
        pltpu.sync_copy(rb[p], g2_hbm.at[cid])

    fire(0, 0)

    def body(i, _):
        j0 = 2 * i
        fire(j0 + 1, 1)
        drain_and_write(j0, 0)

        @pl.when(i < _NCH // 2 - 1)
        def _():
            fire(j0 + 2, 0)

        drain_and_write(j0 + 1, 1)
        return _

    lax.fori_loop(0, _NCH // 2, body, None)


def _sc_gather(xs1, xd1, src_rs, dst_rs):
    mesh = plsc.VectorSubcoreMesh(core_axis_name="c", subcore_axis_name="s")
    f = pl.kernel(
        _gather_body,
        out_type=(
            jax.ShapeDtypeStruct((_NW * _NCH, _CH, _D), jnp.float32),
            jax.ShapeDtypeStruct((_NW * _NCH, _CH, _D), jnp.float32),
        ),
        mesh=mesh,
        scratch_types=[
            pltpu.VMEM((_NCH, _CH), jnp.int32),
            pltpu.VMEM((_NCH, _CH), jnp.int32),
            pltpu.VMEM((_CH, _D), jnp.float32),
            pltpu.VMEM((_CH, _D), jnp.float32),
            pltpu.VMEM((_CH, _D), jnp.float32),
            pltpu.VMEM((_CH, _D), jnp.float32),
            pltpu.SemaphoreType.DMA,
            pltpu.SemaphoreType.DMA,
            pltpu.SemaphoreType.DMA,
            pltpu.SemaphoreType.DMA,
        ],
    )
    g1, g2 = f(xs1, xd1, src_rs, dst_rs)
    return g1.reshape(_E, _D), g2.reshape(_E, _D)


def _edge_body(g1_ref, g2_ref, ea_ref, w1c_ref, b1_ref, w2_ref, b2_ref,
               w3_ref, b3_ref, eg_ref, ebeta_ref, oe_ref, hd_ref, hs_ref):
    ea = ea_ref[...]
    pre1 = (g1_ref[...] + g2_ref[...] + b1_ref[...]
            + jnp.dot(ea, w1c_ref[...], preferred_element_type=jnp.float32))
    h = _silu(pre1)
    h = _silu(jnp.dot(h, w2_ref[...], preferred_element_type=jnp.float32)
              + b2_ref[...])
    e3 = jnp.dot(h, w3_ref[...], preferred_element_type=jnp.float32) + b3_ref[...]
    ne = _ln(e3, eg_ref[...], ebeta_ref[...])
    oe_ref[...] = ne + ea
    hd_ref[...] = ne[:, :_HALF]
    hs_ref[...] = ne[:, _HALF:]


def _edge_mlp(g1, g2, ea, w1c, b1, w2, b2, w3, b3, eg, ebeta):
    grid = _E // _BE
    row = lambda i: (i, 0)
    full = lambda i: (0, 0)
    return pl.pallas_call(
        _edge_body,
        grid=(grid,),
        in_specs=[
            pl.BlockSpec((_BE, _D), row),
            pl.BlockSpec((_BE, _D), row),
            pl.BlockSpec((_BE, _D), row),
            pl.BlockSpec((_D, _D), full),
            pl.BlockSpec((1, _D), full),
            pl.BlockSpec((_D, _D), full),
            pl.BlockSpec((1, _D), full),
            pl.BlockSpec((_D, _D), full),
            pl.BlockSpec((1, _D), full),
            pl.BlockSpec((1, _D), full),
            pl.BlockSpec((1, _D), full),
        ],
        out_specs=[
            pl.BlockSpec((_BE, _D), row),
            pl.BlockSpec((_BE, _HALF), row),
            pl.BlockSpec((_BE, _HALF), row),
        ],
        out_shape=[
            jax.ShapeDtypeStruct((_E, _D), jnp.float32),
            jax.ShapeDtypeStruct((_E, _HALF), jnp.float32),
            jax.ShapeDtypeStruct((_E, _HALF), jnp.float32),
        ],
        interpret=_INTERPRET,
    )(g1, g2, ea, w1c, b1, w2, b2, w3, b3, eg, ebeta)


_NPAD = 10112
_ZCH = _NPAD // _NS
_SCH = 40
_SNCH = _PERW // _SCH
_ZSTG = 64


def _scatter_body(hd_hbm, hs_hbm, dst_hbm, src_hbm, zer_hbm, out_hbm,
                  i0, i1, r0, r1, si0, si1, s0, s1, acc, stg):
    cid_c = lax.axis_index("c")
    sid = lax.axis_index("s")
    wid = sid * _NC + cid_c

    segs = [(k * _ZSTG, _ZSTG) for k in range(_ZCH // _ZSTG)]
    segs.append((_ZCH - _ZCH % _ZSTG, _ZCH % _ZSTG))
    pltpu.sync_copy(zer_hbm, stg)
    for off, sz in segs:
        pltpu.sync_copy(stg.at[pl.ds(0, sz)],
                        acc.at[pl.ds(sid * _ZCH + off, sz)])
    plsc.subcore_barrier()

    r = (r0, r1)
    s = (s0, s1)
    ib = (i0, i1)
    sib = (si0, si1)

    def phase(h_hbm, i_hbm):
        base = wid * _PERW

        def fire(j, p):
            pltpu.make_async_copy(i_hbm.at[pl.ds(base + j * _SCH, _SCH)],
                                  ib[p], sib[p]).start()
            pltpu.make_async_copy(h_hbm.at[wid * _SNCH + j], r[p], s[p]).start()

        def drain_scatter(j, p):
            pltpu.make_async_copy(i_hbm.at[pl.ds(base + j * _SCH, _SCH)],
                                  ib[p], sib[p]).wait()
            pltpu.make_async_copy(h_hbm.at[wid * _SNCH + j], r[p], s[p]).wait()
            pltpu.sync_copy(r[p], acc.at[ib[p]], add=True)

        fire(0, 0)

        def body(i, _):
            j0 = 2 * i
            fire(j0 + 1, 1)
            drain_scatter(j0, 0)

            @pl.when(i < _SNCH // 2 - 1)
            def _():
                fire(j0 + 2, 0)

            drain_scatter(j0 + 1, 1)
            return _

        lax.fori_loop(0, _SNCH // 2, body, None)

    phase(hd_hbm, dst_hbm)
    phase(hs_hbm, src_hbm)

    plsc.subcore_barrier()
    for off, sz in segs:
        pltpu.sync_copy(acc.at[pl.ds(sid * _ZCH + off, sz)],
                        stg.at[pl.ds(0, sz)])
        pltpu.sync_copy(stg.at[pl.ds(0, sz)],
                        out_hbm.at[cid_c, pl.ds(sid * _ZCH + off, sz)])


def _sc_scatter(hd, hs, dst, src):
    mesh = plsc.VectorSubcoreMesh(core_axis_name="c", subcore_axis_name="s")
    zer = jnp.zeros((_ZSTG, _HALF), jnp.float32)
    f = pl.kernel(
        _scatter_body,
        out_type=jax.ShapeDtypeStruct((_NC, _NPAD, _HALF), jnp.float32),
        mesh=mesh,
        scratch_types=[
            pltpu.VMEM((_SCH,), jnp.int32),
            pltpu.VMEM((_SCH,), jnp.int32),
            pltpu.VMEM((_SCH, _HALF), jnp.float32),
            pltpu.VMEM((_SCH, _HALF), jnp.float32),
            pltpu.SemaphoreType.DMA,
            pltpu.SemaphoreType.DMA,
            pltpu.SemaphoreType.DMA,
            pltpu.SemaphoreType.DMA,
            pltpu.VMEM_SHARED((_NPAD, _HALF), jnp.float32),
            pltpu.VMEM((_ZSTG, _HALF), jnp.float32),
        ],
    )
    return f(hd.reshape(_NW * _SNCH, _SCH, _HALF),
             hs.reshape(_NW * _SNCH, _SCH, _HALF),
             dst, src, zer)


def _node_body(x_ref, p0_ref, p1_ref, w1a_ref, w1b_ref, b1_ref, w2_ref,
               b2_ref, w3_ref, b3_ref, ng_ref, nbeta_ref, ox_ref):
    xv = x_ref[...]
    agg = p0_ref[...] + p1_ref[...]
    pre1 = (jnp.dot(xv, w1a_ref[...], preferred_element_type=jnp.float32)
            + jnp.dot(agg, w1b_ref[...], preferred_element_type=jnp.float32)
            + b1_ref[...])
    h = _silu(pre1)
    h = _silu(jnp.dot(h, w2_ref[...], preferred_element_type=jnp.float32)
              + b2_ref[...])
    e3 = jnp.dot(h, w3_ref[...], preferred_element_type=jnp.float32) + b3_ref[...]
    ox_ref[...] = _ln(e3, ng_ref[...], nbeta_ref[...]) + xv


def _node_mlp(x, p0, p1, w1a, w1b, b1, w2, b2, w3, b3, ng, nbeta):
    grid = _N // _BN
    row = lambda i: (i, 0)
    full = lambda i: (0, 0)
    return pl.pallas_call(
        _node_body,
        grid=(grid,),
        in_specs=[
            pl.BlockSpec((_BN, _D), row),
            pl.BlockSpec((_BN, _HALF), row),
            pl.BlockSpec((_BN, _HALF), row),
            pl.BlockSpec((_D, _D), full),
            pl.BlockSpec((_HALF, _D), full),
            pl.BlockSpec((1, _D), full),
            pl.BlockSpec((_D, _D), full),
            pl.BlockSpec((1, _D), full),
            pl.BlockSpec((_D, _D), full),
            pl.BlockSpec((1, _D), full),
            pl.BlockSpec((1, _D), full),
            pl.BlockSpec((1, _D), full),
        ],
        out_specs=pl.BlockSpec((_BN, _D), row),
        out_shape=jax.ShapeDtypeStruct((_N, _D), jnp.float32),
        interpret=_INTERPRET,
    )(x, p0, p1, w1a, w1b, b1, w2, b2, w3, b3, ng, nbeta)


def kernel(x, edge_attr, edge_index,
           eW1, eb1, eW2, eb2, eW3, eb3, eg, ebeta,
           nW1, nb1, nW2, nb2, nW3, nb3, ng, nbeta):
    src = edge_index[0].astype(jnp.int32)
    dst = edge_index[1].astype(jnp.int32)
    w1a, w1b, w1c = eW1[:_D], eW1[_D:2 * _D], eW1[2 * _D:]
    r = lambda v: v.reshape(1, _D)

    xs1, xd1 = _premul(x, w1a, w1b)

    src_rs = src.reshape(_NW, _NCH, _CH)
    dst_rs = dst.reshape(_NW, _NCH, _CH)
    g1, g2 = _sc_gather(xs1, xd1, src_rs, dst_rs)

    out_edge, hd, hs = _edge_mlp(g1, g2, edge_attr, w1c, r(eb1), eW2, r(eb2),
                                 eW3, r(eb3), r(eg), r(ebeta))

    partial = _sc_scatter(hd, hs, dst, src)
    p0 = partial[0, :_N]
    p1 = partial[1, :_N]

    nw1a, nw1b = nW1[:_D], nW1[_D:]
    rh = lambda v: v.reshape(1, _D)
    out_x = _node_mlp(x, p0, p1, nw1a, nw1b, rh(nb1), nW2, rh(nb2),
                      nW3, rh(nb3), rh(ng), rh(nbeta))
    return (out_x, out_edge)


# speedup vs baseline: 3.3493x; 1.2020x over previous
import functools

import jax
import jax.numpy as jnp
from jax import lax
from jax.experimental import pallas as pl
from jax.experimental.pallas import tpu as pltpu
from jax.experimental.pallas import tpu_sc as plsc

_N = 10000
_E = 160000
_D = 128
_HALF = _D // 2

_BN = 1000
_BE = 4000

_NC = 2
_NS = 16
_NW = _NC * _NS
_PERW = _E // _NW
_CH = 100
_NCH = _PERW // _CH

_INTERPRET = False


def _silu(v):
    return v * jax.nn.sigmoid(v)


def _ln(v, g, b):
    j = jnp.full((_D, _D), 1.0 / _D, jnp.float32)
    m = jnp.dot(v, j, preferred_element_type=jnp.float32)
    c = v - m
    var = jnp.dot(c * c, j, preferred_element_type=jnp.float32)
    return c * lax.rsqrt(var + 1e-5) * g + b


def _pre_body(x_ref, wa_ref, wb_ref, oa_ref, ob_ref):
    xv = x_ref[...]
    oa_ref[...] = jnp.dot(xv, wa_ref[...], preferred_element_type=jnp.float32)
    ob_ref[...] = jnp.dot(xv, wb_ref[...], preferred_element_type=jnp.float32)


def _premul(x, w1a, w1b):
    grid = _N // _BN
    return pl.pallas_call(
        _pre_body,
        grid=(grid,),
        in_specs=[
            pl.BlockSpec((_BN, _D), lambda i: (i, 0)),
            pl.BlockSpec((_D, _D), lambda i: (0, 0)),
            pl.BlockSpec((_D, _D), lambda i: (0, 0)),
        ],
        out_specs=[
            pl.BlockSpec((_BN, _D), lambda i: (i, 0)),
            pl.BlockSpec((_BN, _D), lambda i: (i, 0)),
        ],
        out_shape=[
            jax.ShapeDtypeStruct((_N, _D), jnp.float32),
            jax.ShapeDtypeStruct((_N, _D), jnp.float32),
        ],
        interpret=_INTERPRET,
    )(x, w1a, w1b)


def _gather_body(xs1_hbm, xd1_hbm, src_hbm, dst_hbm, g1_hbm, g2_hbm,
                 idx_s, idx_d, ra0, ra1, rb0, rb1,
                 sa0, sa1, sb0, sb1):
    wid = lax.axis_index("s") * _NC + lax.axis_index("c")
    pltpu.sync_copy(src_hbm.at[wid], idx_s)
    pltpu.sync_copy(dst_hbm.at[wid], idx_d)
    ra = (ra0, ra1)
    rb = (rb0, rb1)
    sa = (sa0, sa1)
    sb = (sb0, sb1)

    def fire(j, p):
        pltpu.make_async_copy(xs1_hbm.at[idx_s.at[j]], ra[p], sa[p]).start()
        pltpu.make_async_copy(xd1_hbm.at[idx_d.at[j]], rb[p], sb[p]).start()

    def drain_and_write(j, p):
        pltpu.make_async_copy(xs1_hbm.at[idx_s.at[j]], ra[p], sa[p]).wait()
        pltpu.make_async_copy(xd1_hbm.at[idx_d.at[j]], rb[p], sb[p]).wait()
        cid = wid * _NCH + j
        pltpu.sync_copy(ra[p], g1_hbm.at[cid])
        pltpu.sync_copy(rb[p], g2_hbm.at[cid])

    fire(0, 0)

    def body(i, _):
        j0 = 2 * i
        fire(j0 + 1, 1)
        drain_and_write(j0, 0)

        @pl.when(i < _NCH // 2 - 1)
        def _():
            fire(j0 + 2, 0)

        drain_and_write(j0 + 1, 1)
        return _

    lax.fori_loop(0, _NCH // 2, body, None)


def _sc_gather(xs1, xd1, src_rs, dst_rs):
    mesh = plsc.VectorSubcoreMesh(core_axis_name="c", subcore_axis_name="s")
    f = pl.kernel(
        _gather_body,
        out_type=(
            jax.ShapeDtypeStruct((_NW * _NCH, _CH, _D), jnp.float32),
            jax.ShapeDtypeStruct((_NW * _NCH, _CH, _D), jnp.float32),
        ),
        mesh=mesh,
        scratch_types=[
            pltpu.VMEM((_NCH, _CH), jnp.int32),
            pltpu.VMEM((_NCH, _CH), jnp.int32),
            pltpu.VMEM((_CH, _D), jnp.float32),
            pltpu.VMEM((_CH, _D), jnp.float32),
            pltpu.VMEM((_CH, _D), jnp.float32),
            pltpu.VMEM((_CH, _D), jnp.float32),
            pltpu.SemaphoreType.DMA,
            pltpu.SemaphoreType.DMA,
            pltpu.SemaphoreType.DMA,
            pltpu.SemaphoreType.DMA,
        ],
    )
    g1, g2 = f(xs1, xd1, src_rs, dst_rs)
    return g1.reshape(_E, _D), g2.reshape(_E, _D)


def _edge_body(g1_ref, g2_ref, ea_ref, w1c_ref, b1_ref, w2_ref, b2_ref,
               w3_ref, b3_ref, eg_ref, ebeta_ref, oe_ref, hd_ref, hs_ref):
    ea = ea_ref[...]
    pre1 = (g1_ref[...] + g2_ref[...] + b1_ref[...]
            + jnp.dot(ea, w1c_ref[...], preferred_element_type=jnp.float32))
    h = _silu(pre1)
    h = _silu(jnp.dot(h, w2_ref[...], preferred_element_type=jnp.float32)
              + b2_ref[...])
    e3 = jnp.dot(h, w3_ref[...], preferred_element_type=jnp.float32) + b3_ref[...]
    ne = _ln(e3, eg_ref[...], ebeta_ref[...])
    oe_ref[...] = ne + ea
    hd_ref[...] = ne[:, :_HALF]
    hs_ref[...] = ne[:, _HALF:]


def _edge_mlp(g1, g2, ea, w1c, b1, w2, b2, w3, b3, eg, ebeta):
    grid = _E // _BE
    row = lambda i: (i, 0)
    full = lambda i: (0, 0)
    return pl.pallas_call(
        _edge_body,
        grid=(grid,),
        in_specs=[
            pl.BlockSpec((_BE, _D), row),
            pl.BlockSpec((_BE, _D), row),
            pl.BlockSpec((_BE, _D), row),
            pl.BlockSpec((_D, _D), full),
            pl.BlockSpec((1, _D), full),
            pl.BlockSpec((_D, _D), full),
            pl.BlockSpec((1, _D), full),
            pl.BlockSpec((_D, _D), full),
            pl.BlockSpec((1, _D), full),
            pl.BlockSpec((1, _D), full),
            pl.BlockSpec((1, _D), full),
        ],
        out_specs=[
            pl.BlockSpec((_BE, _D), row),
            pl.BlockSpec((_BE, _HALF), row),
            pl.BlockSpec((_BE, _HALF), row),
        ],
        out_shape=[
            jax.ShapeDtypeStruct((_E, _D), jnp.float32),
            jax.ShapeDtypeStruct((_E, _HALF), jnp.float32),
            jax.ShapeDtypeStruct((_E, _HALF), jnp.float32),
        ],
        interpret=_INTERPRET,
    )(g1, g2, ea, w1c, b1, w2, b2, w3, b3, eg, ebeta)


_NPAD = 10112
_ZCH = _NPAD // _NS
_SCH = 40
_SNCH = _PERW // _SCH
_ZSTG = 64


def _scatter_body(hd_hbm, hs_hbm, dst_hbm, src_hbm, zer_hbm, out_hbm,
                  i0, i1, r0, r1, si0, si1, s0, s1, acc, stg):
    cid_c = lax.axis_index("c")
    sid = lax.axis_index("s")
    wid = sid * _NC + cid_c

    segs = [(k * _ZSTG, _ZSTG) for k in range(_ZCH // _ZSTG)]
    segs.append((_ZCH - _ZCH % _ZSTG, _ZCH % _ZSTG))
    pltpu.sync_copy(zer_hbm, stg)
    for off, sz in segs:
        pltpu.sync_copy(stg.at[pl.ds(0, sz)],
                        acc.at[pl.ds(sid * _ZCH + off, sz)])
    plsc.subcore_barrier()

    r = (r0, r1)
    s = (s0, s1)
    ib = (i0, i1)
    sib = (si0, si1)

    def phase(h_hbm, i_hbm):
        base = wid * _PERW

        def fire(j, p):
            pltpu.make_async_copy(i_hbm.at[pl.ds(base + j * _SCH, _SCH)],
                                  ib[p], sib[p]).start()
            pltpu.make_async_copy(h_hbm.at[wid * _SNCH + j], r[p], s[p]).start()

        def drain_scatter(j, p):
            pltpu.make_async_copy(i_hbm.at[pl.ds(base + j * _SCH, _SCH)],
                                  ib[p], sib[p]).wait()
            pltpu.make_async_copy(h_hbm.at[wid * _SNCH + j], r[p], s[p]).wait()
            pltpu.sync_copy(r[p], acc.at[ib[p]], add=True)

        fire(0, 0)

        def body(i, _):
            j0 = 2 * i
            fire(j0 + 1, 1)
            drain_scatter(j0, 0)

            @pl.when(i < _SNCH // 2 - 1)
            def _():
                fire(j0 + 2, 0)

            drain_scatter(j0 + 1, 1)
            return _

        lax.fori_loop(0, _SNCH // 2, body, None)

    phase(hd_hbm, dst_hbm)
    phase(hs_hbm, src_hbm)

    plsc.subcore_barrier()
    for off, sz in segs:
        pltpu.sync_copy(acc.at[pl.ds(sid * _ZCH + off, sz)],
                        stg.at[pl.ds(0, sz)])
        pltpu.sync_copy(stg.at[pl.ds(0, sz)],
                        out_hbm.at[cid_c, pl.ds(sid * _ZCH + off, sz)])


def _sc_scatter(hd, hs, dst, src):
    mesh = plsc.VectorSubcoreMesh(core_axis_name="c", subcore_axis_name="s")
    zer = jnp.zeros((_ZSTG, _HALF), jnp.float32)
    f = pl.kernel(
        _scatter_body,
        out_type=jax.ShapeDtypeStruct((_NC, _NPAD, _HALF), jnp.float32),
        mesh=mesh,
        scratch_types=[
            pltpu.VMEM((_SCH,), jnp.int32),
            pltpu.VMEM((_SCH,), jnp.int32),
            pltpu.VMEM((_SCH, _HALF), jnp.float32),
            pltpu.VMEM((_SCH, _HALF), jnp.float32),
            pltpu.SemaphoreType.DMA,
            pltpu.SemaphoreType.DMA,
            pltpu.SemaphoreType.DMA,
            pltpu.SemaphoreType.DMA,
            pltpu.VMEM_SHARED((_NPAD, _HALF), jnp.float32),
            pltpu.VMEM((_ZSTG, _HALF), jnp.float32),
        ],
    )
    return f(hd.reshape(_NW * _SNCH, _SCH, _HALF),
             hs.reshape(_NW * _SNCH, _SCH, _HALF),
             dst, src, zer)


def _node_body(x_ref, p0_ref, p1_ref, w1a_ref, w1b_ref, b1_ref, w2_ref,
               b2_ref, w3_ref, b3_ref, ng_ref, nbeta_ref, ox_ref):
    xv = x_ref[...]
    agg = p0_ref[...] + p1_ref[...]
    pre1 = (jnp.dot(xv, w1a_ref[...], preferred_element_type=jnp.float32)
            + jnp.dot(agg, w1b_ref[...], preferred_element_type=jnp.float32)
            + b1_ref[...])
    h = _silu(pre1)
    h = _silu(jnp.dot(h, w2_ref[...], preferred_element_type=jnp.float32)
              + b2_ref[...])
    e3 = jnp.dot(h, w3_ref[...], preferred_element_type=jnp.float32) + b3_ref[...]
    ox_ref[...] = _ln(e3, ng_ref[...], nbeta_ref[...]) + xv


def _node_mlp(x, p0, p1, w1a, w1b, b1, w2, b2, w3, b3, ng, nbeta):
    grid = _N // _BN
    row = lambda i: (i, 0)
    full = lambda i: (0, 0)
    return pl.pallas_call(
        _node_body,
        grid=(grid,),
        in_specs=[
            pl.BlockSpec((_BN, _D), row),
            pl.BlockSpec((_BN, _HALF), row),
            pl.BlockSpec((_BN, _HALF), row),
            pl.BlockSpec((_D, _D), full),
            pl.BlockSpec((_HALF, _D), full),
            pl.BlockSpec((1, _D), full),
            pl.BlockSpec((_D, _D), full),
            pl.BlockSpec((1, _D), full),
            pl.BlockSpec((_D, _D), full),
            pl.BlockSpec((1, _D), full),
            pl.BlockSpec((1, _D), full),
            pl.BlockSpec((1, _D), full),
        ],
        out_specs=pl.BlockSpec((_BN, _D), row),
        out_shape=jax.ShapeDtypeStruct((_N, _D), jnp.float32),
        interpret=_INTERPRET,
    )(x, p0, p1, w1a, w1b, b1, w2, b2, w3, b3, ng, nbeta)


def kernel(x, edge_attr, edge_index,
           eW1, eb1, eW2, eb2, eW3, eb3, eg, ebeta,
           nW1, nb1, nW2, nb2, nW3, nb3, ng, nbeta):
    src = edge_index[0].astype(jnp.int32)
    dst = edge_index[1].astype(jnp.int32)
    w1a, w1b, w1c = eW1[:_D], eW1[_D:2 * _D], eW1[2 * _D:]
    r = lambda v: v.reshape(1, _D)

    xs1, xd1 = _premul(x, w1a, w1b)

    src_rs = src.reshape(_NW, _NCH, _CH)
    dst_rs = dst.reshape(_NW, _NCH, _CH)
    g1, g2 = _sc_gather(xs1, xd1, src_rs, dst_rs)

    out_edge, hd, hs = _edge_mlp(g1, g2, edge_attr, w1c, r(eb1), eW2, r(eb2),
                                 eW3, r(eb3), r(eg), r(ebeta))

    partial = _sc_scatter(hd, hs, dst, src)
    p0 = partial[0, :_N]
    p1 = partial[1, :_N]

    nw1a, nw1b = nW1[:_D], nW1[_D:]
    rh = lambda v: v.reshape(1, _D)
    out_x = _node_mlp(x, p0, p1, nw1a, nw1b, rh(nb1), nW2, rh(nb2),
                      nW3, rh(nb3), rh(ng), rh(nbeta))
    return (out_x, out_edge)
